# baseline (speedup 1.0000x reference)
import jax
import jax.numpy as jnp
from jax.experimental import pallas as pl
from jax.experimental.pallas import tpu as pltpu

_TOKENS = 8192
_HIDDEN = 2048
_EXPERTS = 16
_CHUNK = 256
_NBUF = 16
_NCHUNKS = _TOKENS // _CHUNK


def _gating_body(x_hbm, w_ref, b_ref, wout_ref, idx_ref, ent_ref, buf, wt, sem):
    def start(c, slot):
        pltpu.make_async_copy(
            x_hbm.at[pl.ds(c * _CHUNK, _CHUNK), :],
            buf.at[slot],
            sem.at[slot],
        ).start()

    for s in range(min(_NBUF, _NCHUNKS)):
        start(s, s)

    wt[...] = w_ref[...].T
    wout_ref[...] = jnp.ones((_TOKENS, 1), jnp.float32)
    bias = b_ref[...].reshape(1, _EXPERTS)

    def step(c, acc):
        slot = jax.lax.rem(c, _NBUF)
        pltpu.make_async_copy(
            x_hbm.at[pl.ds(c * _CHUNK, _CHUNK), :],
            buf.at[slot],
            sem.at[slot],
        ).wait()
        nxt = c + _NBUF

        @pl.when(nxt < _NCHUNKS)
        def _():
            start(nxt, slot)

        logits = jnp.dot(buf[slot], wt[...],
                         preferred_element_type=jnp.float32) + bias

        m = jnp.max(logits, axis=-1, keepdims=True)
        e = jnp.exp(logits - m)
        s = jnp.sum(e, axis=-1, keepdims=True)
        ent_tok = (jnp.sum(e * (logits - m), axis=-1, keepdims=True) / s
                   - jnp.log(s))

        lane = jax.lax.broadcasted_iota(jnp.int32, logits.shape, 1)
        idx = jnp.min(jnp.where(logits == m, lane, _EXPERTS), axis=-1)

        idx_ref[pl.ds(c * _CHUNK, _CHUNK)] = idx
        return acc + jnp.sum(ent_tok)

    total = jax.lax.fori_loop(0, _NCHUNKS, step, jnp.float32(0.0))
    ent_ref[...] = total * (1.0 / _TOKENS)


@jax.jit
def kernel(x, W, b):
    return pl.pallas_call(
        _gating_body,
        in_specs=[
            pl.BlockSpec(memory_space=pltpu.MemorySpace.HBM),
            pl.BlockSpec(memory_space=pltpu.VMEM),
            pl.BlockSpec(memory_space=pltpu.VMEM),
        ],
        out_specs=[
            pl.BlockSpec(memory_space=pltpu.VMEM),
            pl.BlockSpec(memory_space=pltpu.VMEM),
            pl.BlockSpec(memory_space=pltpu.SMEM),
        ],
        out_shape=[
            jax.ShapeDtypeStruct((_TOKENS, 1), jnp.float32),
            jax.ShapeDtypeStruct((_TOKENS,), jnp.int32),
            jax.ShapeDtypeStruct((), jnp.float32),
        ],
        scratch_shapes=[
            pltpu.VMEM((_NBUF, _CHUNK, _HIDDEN), jnp.float32),
            pltpu.VMEM((_HIDDEN, _EXPERTS), jnp.float32),
            pltpu.SemaphoreType.DMA((_NBUF,)),
        ],
    )(x, W, b)


# speedup vs baseline: 1.0871x; 1.0871x over previous
import jax
import jax.numpy as jnp
from jax.experimental import pallas as pl
from jax.experimental.pallas import tpu as pltpu

_TOKENS = 8192
_HIDDEN = 2048
_EXPERTS = 16
_CHUNK = 512
_NBUF = 8
_NCHUNKS = _TOKENS // _CHUNK


def _gating_body(x_hbm, w_ref, b_ref, wout_ref, idx_ref, ent_ref, buf, wt, sem):
    def start(c, slot):
        pltpu.make_async_copy(
            x_hbm.at[pl.ds(c * _CHUNK, _CHUNK), :],
            buf.at[slot],
            sem.at[slot],
        ).start()

    for s in range(min(_NBUF, _NCHUNKS)):
        start(s, s)

    wt[...] = w_ref[...].T
    wout_ref[...] = jnp.ones((_TOKENS, 1), jnp.float32)
    bias = b_ref[...].reshape(1, _EXPERTS)

    def step(c, acc):
        slot = jax.lax.rem(c, _NBUF)
        pltpu.make_async_copy(
            x_hbm.at[pl.ds(c * _CHUNK, _CHUNK), :],
            buf.at[slot],
            sem.at[slot],
        ).wait()
        nxt = c + _NBUF

        @pl.when(nxt < _NCHUNKS)
        def _():
            start(nxt, slot)

        logits = jnp.dot(buf[slot], wt[...],
                         preferred_element_type=jnp.float32) + bias

        m = jnp.max(logits, axis=-1, keepdims=True)
        e = jnp.exp(logits - m)
        s = jnp.sum(e, axis=-1, keepdims=True)
        ent_tok = (jnp.sum(e * (logits - m), axis=-1, keepdims=True) / s
                   - jnp.log(s))

        lane = jax.lax.broadcasted_iota(jnp.int32, logits.shape, 1)
        idx = jnp.min(jnp.where(logits == m, lane, _EXPERTS), axis=-1)

        idx_ref[pl.ds(c * _CHUNK, _CHUNK)] = idx
        return acc + jnp.sum(ent_tok)

    total = jax.lax.fori_loop(0, _NCHUNKS, step, jnp.float32(0.0))
    ent_ref[...] = total * (1.0 / _TOKENS)


@jax.jit
def kernel(x, W, b):
    return pl.pallas_call(
        _gating_body,
        in_specs=[
            pl.BlockSpec(memory_space=pltpu.MemorySpace.HBM),
            pl.BlockSpec(memory_space=pltpu.VMEM),
            pl.BlockSpec(memory_space=pltpu.VMEM),
        ],
        out_specs=[
            pl.BlockSpec(memory_space=pltpu.VMEM),
            pl.BlockSpec(memory_space=pltpu.VMEM),
            pl.BlockSpec(memory_space=pltpu.SMEM),
        ],
        out_shape=[
            jax.ShapeDtypeStruct((_TOKENS, 1), jnp.float32),
            jax.ShapeDtypeStruct((_TOKENS,), jnp.int32),
            jax.ShapeDtypeStruct((), jnp.float32),
        ],
        scratch_shapes=[
            pltpu.VMEM((_NBUF, _CHUNK, _HIDDEN), jnp.float32),
            pltpu.VMEM((_HIDDEN, _EXPERTS), jnp.float32),
            pltpu.SemaphoreType.DMA((_NBUF,)),
        ],
    )(x, W, b)
